# baseline (speedup 1.0000x reference)
import functools

import jax
import jax.numpy as jnp
from jax import lax
from jax.experimental import pallas as pl
from jax.experimental.pallas import tpu as pltpu
from jax.experimental.pallas import tpu_sc as plsc

N = 640000
D = 128
B = 64

NC = 2
NS = 16
NW = NC * NS
L = 16

N_SC = 384000
N_TC = N - N_SC

ROWS_PER_TILE = N_SC // NW
CHUNK = 400
NCHUNK = ROWS_PER_TILE // CHUNK
GROUPS = CHUNK // L

RB = 2048
NB = N_TC // RB


def _sc_partials(features_flat, batch_ids):
  mesh = plsc.VectorSubcoreMesh(core_axis_name="c", subcore_axis_name="s")

  @functools.partial(
      pl.kernel,
      mesh=mesh,
      out_type=[
          jax.ShapeDtypeStruct((NW * B * D,), jnp.float32),
          jax.ShapeDtypeStruct((NW * B * L,), jnp.float32),
      ],
      scratch_types=[
          pltpu.VMEM((CHUNK * D,), jnp.float32),
          pltpu.VMEM((CHUNK * D,), jnp.float32),
          pltpu.VMEM((CHUNK,), jnp.int32),
          pltpu.VMEM((CHUNK,), jnp.int32),
          pltpu.VMEM((B * D,), jnp.float32),
          pltpu.VMEM((B * L,), jnp.float32),
          pltpu.SemaphoreType.DMA,
          pltpu.SemaphoreType.DMA,
      ],
  )
  def k(feat_hbm, ids_hbm, sum_hbm, cnt_hbm,
        fbuf0, fbuf1, ibuf0, ibuf1, acc, cnt, sem0, sem1):
    wid = lax.axis_index("s") * NC + lax.axis_index("c")
    base = wid * ROWS_PER_TILE

    fbufs = (fbuf0, fbuf1)
    ibufs = (ibuf0, ibuf1)
    sems = (sem0, sem1)

    zeros = jnp.zeros((L,), jnp.float32)
    def _z(i, _):
      acc[pl.ds(i * L, L)] = zeros
      return 0
    lax.fori_loop(0, B * D // L, _z, 0)
    def _zc(i, _):
      cnt[pl.ds(i * L, L)] = zeros
      return 0
    lax.fori_loop(0, B, _zc, 0)

    ones = jnp.ones((L,), jnp.float32)

    def issue(ci, b):
      start = base + ci * CHUNK
      pltpu.async_copy(feat_hbm.at[pl.ds(start * D, CHUNK * D)],
                       fbufs[b], sems[b])
      pltpu.async_copy(ids_hbm.at[pl.ds(start, CHUNK)], ibufs[b], sems[b])

    def drain(b):
      pltpu.make_async_copy(
          feat_hbm.at[pl.ds(0, CHUNK * D)], fbufs[b], sems[b]).wait()
      pltpu.make_async_copy(
          ids_hbm.at[pl.ds(0, CHUNK)], ibufs[b], sems[b]).wait()

    def process(b):
      fbuf = fbufs[b]
      ibuf = ibufs[b]
      head = ibuf[pl.ds(0, L)]
      tail = ibuf[pl.ds(CHUNK - L, L)]
      first = head[0]
      uniform = first == tail[L - 1]

      @pl.when(uniform)
      def _fast():
        def rb(r, accs):
          off = r * D
          return tuple(accs[c] + fbuf[pl.ds(off + c * L, L)]
                       for c in range(D // L))
        accs = (zeros,) * (D // L)
        def rb8(q, accs):
          for u in range(8):
            accs = rb(8 * q + u, accs)
          return accs
        accs = lax.fori_loop(0, CHUNK // 8, rb8, accs)
        dst = first * D
        for c in range(D // L):
          plsc.addupdate(acc.at[pl.ds(dst + c * L, L)], accs[c])
        plsc.addupdate(cnt.at[pl.ds(first * L, L)], ones * float(CHUNK))

      @pl.when(jnp.logical_not(uniform))
      def _slow():
        def group_body(g, _):
          segs = ibuf[pl.ds(g * L, L)]
          dsts = segs * D
          csts = segs * L
          for r in range(L):
            off = (g * L + r) * D
            dst = dsts[r]
            plsc.addupdate(cnt.at[pl.ds(csts[r], L)], ones)
            for c in range(D // L):
              v = fbuf[pl.ds(off + c * L, L)]
              plsc.addupdate(acc.at[pl.ds(dst + c * L, L)], v)
          return 0
        lax.fori_loop(0, GROUPS, group_body, 0)

    issue(0, 0)
    def chunk_pair(p, _):
      ci = 2 * p
      issue(ci + 1, 1)
      drain(0)
      process(0)
      @pl.when(ci + 2 < NCHUNK)
      def _():
        issue(ci + 2, 0)
      drain(1)
      process(1)
      return 0
    lax.fori_loop(0, NCHUNK // 2, chunk_pair, 0)

    pltpu.sync_copy(acc, sum_hbm.at[pl.ds(wid * B * D, B * D)])
    pltpu.sync_copy(cnt, cnt_hbm.at[pl.ds(wid * B * L, B * L)])

  return k(features_flat, batch_ids)


def _tc_body(ids_ref, feat_ref, sum_ref, cnt_ref):
  i = pl.program_id(0)
  ids = ids_ref[0, 0, :]
  first = ids[0]
  uniform = first == ids[RB - 1]

  @pl.when(i == 0)
  def _():
    sum_ref[...] = jnp.zeros((B, D), jnp.float32)
    cnt_ref[...] = jnp.zeros((B, D), jnp.float32)

  @pl.when(uniform)
  def _fast():
    colsum = jnp.sum(feat_ref[...], axis=0)
    mask = (lax.broadcasted_iota(jnp.int32, (B, 1), 0) == first).astype(
        jnp.float32)
    sum_ref[...] += mask * colsum[None, :]
    cnt_ref[...] += mask * float(RB)

  @pl.when(jnp.logical_not(uniform))
  def _slow():
    seg_iota = lax.broadcasted_iota(jnp.int32, (B, RB), 0)
    oh = (ids[None, :] == seg_iota).astype(jnp.float32)
    sum_ref[...] += jnp.dot(oh, feat_ref[...],
                            preferred_element_type=jnp.float32)
    cnt_ref[...] += jnp.broadcast_to(jnp.sum(oh, axis=1)[:, None], (B, D))


def _tc_partials(feat_tc, ids_tc):
  return pl.pallas_call(
      _tc_body,
      grid=(NB,),
      in_specs=[
          pl.BlockSpec((1, 1, RB), lambda i: (i, 0, 0)),
          pl.BlockSpec((RB, D), lambda i: (i, 0)),
      ],
      out_specs=[
          pl.BlockSpec((B, D), lambda i: (0, 0)),
          pl.BlockSpec((B, D), lambda i: (0, 0)),
      ],
      out_shape=[
          jax.ShapeDtypeStruct((B, D), jnp.float32),
          jax.ShapeDtypeStruct((B, D), jnp.float32),
      ],
  )(ids_tc, feat_tc)


def _combine_kernel(sum_ref, cnt_ref, tsum_ref, tcnt_ref, out_ref):
  s = jnp.sum(sum_ref[...], axis=0) + tsum_ref[...]
  c = jnp.sum(cnt_ref[:, :, 0], axis=0) + tcnt_ref[:, 0]
  out_ref[...] = s / jnp.maximum(c, 1.0)[:, None]


def _combine(sums, cnts, tsum, tcnt):
  return pl.pallas_call(
      _combine_kernel,
      out_shape=jax.ShapeDtypeStruct((B, D), jnp.float32),
  )(sums, cnts, tsum, tcnt)


@jax.jit
def kernel(features, batch_ids):
  ids = batch_ids.astype(jnp.int32)
  feat_sc = features[:N_SC].reshape((N_SC * D,))
  sums, cnts = _sc_partials(feat_sc, ids[:N_SC])
  tsum, tcnt = _tc_partials(features[N_SC:], ids[N_SC:].reshape((NB, 1, RB)))
  return _combine(sums.reshape((NW, B, D)), cnts.reshape((NW, B, L)),
                  tsum, tcnt)


# speedup vs baseline: 2.4244x; 2.4244x over previous
import functools

import jax
import jax.numpy as jnp
from jax import lax
from jax.experimental import pallas as pl
from jax.experimental.pallas import tpu as pltpu
from jax.experimental.pallas import tpu_sc as plsc

N = 640000
D = 128
B = 64

NC = 2
NS = 16
NW = NC * NS
L = 16

N_SC = 384000
N_TC = N - N_SC

ROWS_PER_TILE = N_SC // NW
CHUNK = 400
NCHUNK = ROWS_PER_TILE // CHUNK
GROUPS = CHUNK // L

RB = 2000
NB = N_TC // RB
TC_OFF = N_SC // RB


def _sc_partials(features_flat, batch_ids):
  mesh = plsc.VectorSubcoreMesh(core_axis_name="c", subcore_axis_name="s")

  @functools.partial(
      pl.kernel,
      mesh=mesh,
      out_type=[
          jax.ShapeDtypeStruct((NW * B * D,), jnp.float32),
          jax.ShapeDtypeStruct((NW * B * L,), jnp.float32),
      ],
      scratch_types=[
          pltpu.VMEM((CHUNK * D,), jnp.float32),
          pltpu.VMEM((CHUNK * D,), jnp.float32),
          pltpu.VMEM((CHUNK,), jnp.int32),
          pltpu.VMEM((CHUNK,), jnp.int32),
          pltpu.VMEM((B * D,), jnp.float32),
          pltpu.VMEM((B * L,), jnp.float32),
          pltpu.SemaphoreType.DMA,
          pltpu.SemaphoreType.DMA,
      ],
  )
  def k(feat_hbm, ids_hbm, sum_hbm, cnt_hbm,
        fbuf0, fbuf1, ibuf0, ibuf1, acc, cnt, sem0, sem1):
    wid = lax.axis_index("s") * NC + lax.axis_index("c")
    base = wid * ROWS_PER_TILE

    fbufs = (fbuf0, fbuf1)
    ibufs = (ibuf0, ibuf1)
    sems = (sem0, sem1)

    zeros = jnp.zeros((L,), jnp.float32)
    def _z(i, _):
      acc[pl.ds(i * L, L)] = zeros
      return 0
    lax.fori_loop(0, B * D // L, _z, 0)
    def _zc(i, _):
      cnt[pl.ds(i * L, L)] = zeros
      return 0
    lax.fori_loop(0, B, _zc, 0)

    ones = jnp.ones((L,), jnp.float32)

    def issue(ci, b):
      start = base + ci * CHUNK
      pltpu.async_copy(feat_hbm.at[pl.ds(start * D, CHUNK * D)],
                       fbufs[b], sems[b])
      pltpu.async_copy(ids_hbm.at[pl.ds(start, CHUNK)], ibufs[b], sems[b])

    def drain(b):
      pltpu.make_async_copy(
          feat_hbm.at[pl.ds(0, CHUNK * D)], fbufs[b], sems[b]).wait()
      pltpu.make_async_copy(
          ids_hbm.at[pl.ds(0, CHUNK)], ibufs[b], sems[b]).wait()

    def process(b):
      fbuf = fbufs[b]
      ibuf = ibufs[b]
      head = ibuf[pl.ds(0, L)]
      tail = ibuf[pl.ds(CHUNK - L, L)]
      first = head[0]
      uniform = first == tail[L - 1]

      @pl.when(uniform)
      def _fast():
        def rb(r, accs):
          off = r * D
          return tuple(accs[c] + fbuf[pl.ds(off + c * L, L)]
                       for c in range(D // L))
        accs = (zeros,) * (D // L)
        def rb8(q, accs):
          for u in range(8):
            accs = rb(8 * q + u, accs)
          return accs
        accs = lax.fori_loop(0, CHUNK // 8, rb8, accs)
        dst = first * D
        for c in range(D // L):
          plsc.addupdate(acc.at[pl.ds(dst + c * L, L)], accs[c])
        plsc.addupdate(cnt.at[pl.ds(first * L, L)], ones * float(CHUNK))

      @pl.when(jnp.logical_not(uniform))
      def _slow():
        def group_body(g, _):
          segs = ibuf[pl.ds(g * L, L)]
          dsts = segs * D
          csts = segs * L
          for r in range(L):
            off = (g * L + r) * D
            dst = dsts[r]
            plsc.addupdate(cnt.at[pl.ds(csts[r], L)], ones)
            for c in range(D // L):
              v = fbuf[pl.ds(off + c * L, L)]
              plsc.addupdate(acc.at[pl.ds(dst + c * L, L)], v)
          return 0
        lax.fori_loop(0, GROUPS, group_body, 0)

    issue(0, 0)
    def chunk_pair(p, _):
      ci = 2 * p
      issue(ci + 1, 1)
      drain(0)
      process(0)
      @pl.when(ci + 2 < NCHUNK)
      def _():
        issue(ci + 2, 0)
      drain(1)
      process(1)
      return 0
    lax.fori_loop(0, NCHUNK // 2, chunk_pair, 0)

    pltpu.sync_copy(acc, sum_hbm.at[pl.ds(wid * B * D, B * D)])
    pltpu.sync_copy(cnt, cnt_hbm.at[pl.ds(wid * B * L, B * L)])

  return k(features_flat, batch_ids)


def _tc_body(ids_ref, feat_ref, sum_ref, cnt_ref):
  i = pl.program_id(0)
  ids = ids_ref[0, 0, :]
  first = ids[0]
  uniform = first == ids[RB - 1]

  @pl.when(i == 0)
  def _():
    sum_ref[...] = jnp.zeros((B, D), jnp.float32)
    cnt_ref[...] = jnp.zeros((B, D), jnp.float32)

  @pl.when(uniform)
  def _fast():
    colsum = jnp.sum(feat_ref[...], axis=0)
    mask = (lax.broadcasted_iota(jnp.int32, (B, 1), 0) == first).astype(
        jnp.float32)
    sum_ref[...] += mask * colsum[None, :]
    cnt_ref[...] += mask * float(RB)

  @pl.when(jnp.logical_not(uniform))
  def _slow():
    seg_iota = lax.broadcasted_iota(jnp.int32, (B, RB), 0)
    oh = (ids[None, :] == seg_iota).astype(jnp.float32)
    sum_ref[...] += jnp.dot(oh, feat_ref[...],
                            preferred_element_type=jnp.float32)
    cnt_ref[...] += jnp.broadcast_to(jnp.sum(oh, axis=1)[:, None], (B, D))


def _tc_partials(feat, ids3d):
  return pl.pallas_call(
      _tc_body,
      grid=(NB,),
      in_specs=[
          pl.BlockSpec((1, 1, RB), lambda i: (i + TC_OFF, 0, 0)),
          pl.BlockSpec((RB, D), lambda i: (i + TC_OFF, 0)),
      ],
      out_specs=[
          pl.BlockSpec((B, D), lambda i: (0, 0)),
          pl.BlockSpec((B, D), lambda i: (0, 0)),
      ],
      out_shape=[
          jax.ShapeDtypeStruct((B, D), jnp.float32),
          jax.ShapeDtypeStruct((B, D), jnp.float32),
      ],
  )(ids3d, feat)


def _combine_kernel(sum_ref, cnt_ref, tsum_ref, tcnt_ref, out_ref):
  s = jnp.sum(sum_ref[...], axis=0) + tsum_ref[...]
  c = jnp.sum(cnt_ref[:, :, 0], axis=0) + tcnt_ref[:, 0]
  out_ref[...] = s / jnp.maximum(c, 1.0)[:, None]


def _combine(sums, cnts, tsum, tcnt):
  return pl.pallas_call(
      _combine_kernel,
      out_shape=jax.ShapeDtypeStruct((B, D), jnp.float32),
  )(sums, cnts, tsum, tcnt)


@jax.jit
def kernel(features, batch_ids):
  ids = batch_ids.astype(jnp.int32)
  sums, cnts = _sc_partials(features.reshape((N * D,)), ids)
  tsum, tcnt = _tc_partials(features, ids.reshape((N // RB, 1, RB)))
  return _combine(sums.reshape((NW, B, D)), cnts.reshape((NW, B, L)),
                  tsum, tcnt)
